# baseline (speedup 1.0000x reference)
import functools

import jax
import jax.numpy as jnp
from jax.experimental import pallas as pl

_WPAD = 40


def _rpn_body(x_ref, w1_ref, b1_ref, w2_ref, b2_ref, o_ref, *, rt):
    r0 = pl.multiple_of(pl.program_id(1) * rt, 8)
    acc = jnp.zeros((rt, w1_ref.shape[2]), jnp.float32)
    for ky in range(3):
        xs = x_ref[0, pl.ds(r0 + ky * _WPAD, rt), :]
        acc = acc + jnp.dot(xs, w1_ref[ky], preferred_element_type=jnp.float32)
    h = jnp.maximum(acc + b1_ref[0].astype(jnp.float32), 0.0).astype(jnp.bfloat16)
    out = jnp.dot(h, w2_ref[...], preferred_element_type=jnp.float32)
    o_ref[0] = out + b2_ref[0].astype(jnp.float32)


def kernel(feats, W1, b1, W2, b2):
    B, C, H, W = feats.shape
    dim = W1.shape[0]
    co = W2.shape[0]
    Hp = H + 2

    RT = 256
    nR = -(-(H * _WPAD) // RT)
    Rpad = nR * RT + 2 * _WPAD
    CO2 = 128

    x = jnp.transpose(feats, (0, 2, 3, 1)).astype(jnp.bfloat16)
    xp = jnp.pad(x, ((0, 0), (1, 1), (1, _WPAD - W - 1), (0, 0)))
    xf = xp.reshape(B, Hp * _WPAD, C)
    xf = jnp.pad(xf, ((0, 0), (0, Rpad + 2 - Hp * _WPAD), (0, 0)))
    xcat = jnp.concatenate([xf[:, k:k + Rpad, :] for k in range(3)], axis=2)
    w1 = jnp.transpose(W1, (2, 3, 1, 0)).reshape(3, 3 * C, dim)
    w1 = w1.astype(jnp.bfloat16)
    w2 = jnp.pad(W2[:, :, 0, 0].T, ((0, 0), (0, CO2 - co))).astype(jnp.bfloat16)
    b1r = b1.reshape(1, dim)
    b2r = jnp.pad(b2, (0, CO2 - co)).reshape(1, CO2)

    body = functools.partial(_rpn_body, rt=RT)
    out = pl.pallas_call(
        body,
        grid=(B, nR),
        in_specs=[
            pl.BlockSpec((1, Rpad, 3 * C), lambda b, r: (b, 0, 0)),
            pl.BlockSpec((3, 3 * C, dim), lambda b, r: (0, 0, 0)),
            pl.BlockSpec((1, dim), lambda b, r: (0, 0)),
            pl.BlockSpec((dim, CO2), lambda b, r: (0, 0)),
            pl.BlockSpec((1, CO2), lambda b, r: (0, 0)),
        ],
        out_specs=pl.BlockSpec((1, RT, CO2), lambda b, r: (b, r, 0)),
        out_shape=jax.ShapeDtypeStruct((B, nR * RT, CO2), jnp.float32),
    )(xcat, w1, b1r, w2, b2r)

    out = out[:, :H * _WPAD, :].reshape(B, H, _WPAD, CO2)[:, :, :W, :co]
    return out.reshape(B, H, W, co // 6, 6)


# speedup vs baseline: 1.3677x; 1.3677x over previous
import functools

import jax
import jax.numpy as jnp
from jax.experimental import pallas as pl

_WPAD = 40


def _rpn_body(x_ref, w1_ref, b1_ref, w2_ref, b2_ref, o_ref, *, rt):
    c = w1_ref.shape[1]
    r0 = pl.multiple_of(pl.program_id(1) * rt, 8)
    acc = jnp.zeros((rt, w1_ref.shape[2]), jnp.float32)
    for ky in range(3):
        xw = x_ref[0, pl.ds(r0 + ky * _WPAD, rt + 8), :]
        for kx in range(3):
            xs = jax.lax.slice(xw, (kx, 0), (kx + rt, c))
            acc = acc + jnp.dot(xs, w1_ref[3 * ky + kx],
                                preferred_element_type=jnp.float32)
    h = jnp.maximum(acc + b1_ref[0].astype(jnp.float32), 0.0).astype(jnp.bfloat16)
    out = jnp.dot(h, w2_ref[...], preferred_element_type=jnp.float32)
    o_ref[0] = out + b2_ref[0].astype(jnp.float32)


def kernel(feats, W1, b1, W2, b2):
    B, C, H, W = feats.shape
    dim = W1.shape[0]
    co = W2.shape[0]
    Hp = H + 2

    RT = 256
    nR = -(-(H * _WPAD) // RT)
    Rpad = nR * RT + 2 * _WPAD + 8
    CO2 = 128

    x = jnp.transpose(feats, (0, 2, 3, 1)).astype(jnp.bfloat16)
    xp = jnp.pad(x, ((0, 0), (1, 1), (1, _WPAD - W - 1), (0, 0)))
    xf = xp.reshape(B, Hp * _WPAD, C)
    xf = jnp.pad(xf, ((0, 0), (0, Rpad - Hp * _WPAD), (0, 0)))
    w1 = jnp.transpose(W1, (2, 3, 1, 0)).reshape(9, C, dim)
    w1 = w1.astype(jnp.bfloat16)
    w2 = jnp.pad(W2[:, :, 0, 0].T, ((0, 0), (0, CO2 - co))).astype(jnp.bfloat16)
    b1r = b1.reshape(1, dim)
    b2r = jnp.pad(b2, (0, CO2 - co)).reshape(1, CO2)

    body = functools.partial(_rpn_body, rt=RT)
    out = pl.pallas_call(
        body,
        grid=(B, nR),
        in_specs=[
            pl.BlockSpec((1, Rpad, C), lambda b, r: (b, 0, 0)),
            pl.BlockSpec((9, C, dim), lambda b, r: (0, 0, 0)),
            pl.BlockSpec((1, dim), lambda b, r: (0, 0)),
            pl.BlockSpec((dim, CO2), lambda b, r: (0, 0)),
            pl.BlockSpec((1, CO2), lambda b, r: (0, 0)),
        ],
        out_specs=pl.BlockSpec((1, RT, CO2), lambda b, r: (b, r, 0)),
        out_shape=jax.ShapeDtypeStruct((B, nR * RT, CO2), jnp.float32),
    )(xf, w1, b1r, w2, b2r)

    out = out[:, :H * _WPAD, :].reshape(B, H, _WPAD, CO2)[:, :, :W, :co]
    return out.reshape(B, H, W, co // 6, 6)


# speedup vs baseline: 1.4168x; 1.0359x over previous
import functools

import jax
import jax.numpy as jnp
from jax.experimental import pallas as pl

_WPAD = 40


def _rpn_body(x_ref, w1_ref, b1_ref, w2_ref, b2_ref, o_ref, *, rt):
    c = w1_ref.shape[1]
    r0 = pl.multiple_of(pl.program_id(1) * rt, 8)
    acc = jnp.zeros((rt, w1_ref.shape[2]), jnp.float32)
    for ky in range(3):
        xw = x_ref[0, pl.ds(r0 + ky * _WPAD, rt + 8), :]
        for kx in range(3):
            xs = jax.lax.slice(xw, (kx, 0), (kx + rt, c))
            acc = acc + jnp.dot(xs, w1_ref[3 * ky + kx],
                                preferred_element_type=jnp.float32)
    h = jnp.maximum(acc + b1_ref[0].astype(jnp.float32), 0.0).astype(jnp.bfloat16)
    out = jnp.dot(h, w2_ref[...], preferred_element_type=jnp.float32)
    o_ref[0] = out + b2_ref[0].astype(jnp.float32)


def kernel(feats, W1, b1, W2, b2):
    B, C, H, W = feats.shape
    dim = W1.shape[0]
    co = W2.shape[0]
    Hp = H + 2

    RT = 512
    nR = -(-(H * _WPAD) // RT)
    Hpad = -(-(nR * RT + 2 * _WPAD + 8) // _WPAD)
    Rpad = Hpad * _WPAD
    CO2 = 128

    xp = jnp.pad(feats.astype(jnp.bfloat16),
                 ((0, 0), (0, 0), (1, Hpad - H - 1), (1, _WPAD - W - 1)))
    xf = jnp.transpose(xp.reshape(B, C, Rpad), (0, 2, 1))
    w1 = jnp.transpose(W1, (2, 3, 1, 0)).reshape(9, C, dim)
    w1 = w1.astype(jnp.bfloat16)
    w2 = jnp.pad(W2[:, :, 0, 0].T, ((0, 0), (0, CO2 - co))).astype(jnp.bfloat16)
    b1r = b1.reshape(1, dim)
    b2r = jnp.pad(b2, (0, CO2 - co)).reshape(1, CO2)

    body = functools.partial(_rpn_body, rt=RT)
    out = pl.pallas_call(
        body,
        grid=(B, nR),
        in_specs=[
            pl.BlockSpec((1, Rpad, C), lambda b, r: (b, 0, 0)),
            pl.BlockSpec((9, C, dim), lambda b, r: (0, 0, 0)),
            pl.BlockSpec((1, dim), lambda b, r: (0, 0)),
            pl.BlockSpec((dim, CO2), lambda b, r: (0, 0)),
            pl.BlockSpec((1, CO2), lambda b, r: (0, 0)),
        ],
        out_specs=pl.BlockSpec((1, RT, CO2), lambda b, r: (b, r, 0)),
        out_shape=jax.ShapeDtypeStruct((B, nR * RT, CO2), jnp.float32),
    )(xf, w1, b1r, w2, b2r)

    out = out[:, :H * _WPAD, :].reshape(B, H, _WPAD, CO2)[:, :, :W, :co]
    return out.reshape(B, H, W, co // 6, 6)


# speedup vs baseline: 1.4566x; 1.0281x over previous
import functools

import jax
import jax.numpy as jnp
from jax.experimental import pallas as pl
from jax.experimental.pallas import tpu as pltpu

_WPAD = 40


def _rpn_body(xt_ref, w1_ref, b1_ref, w2_ref, b2_ref, o_ref, xs_pad, *,
              rt, h, w):
    c = w1_ref.shape[1]
    r = pl.program_id(1)

    @pl.when(r == 0)
    def _build_padded_slab():
        xs_pad[...] = jnp.zeros_like(xs_pad)
        for y in range(h):
            xs_pad[pl.ds((y + 1) * _WPAD + 1, w), :] = (
                xt_ref[0, pl.ds(y * w, w), :].astype(jnp.bfloat16))

    r0 = pl.multiple_of(r * rt, 16)
    acc = jnp.zeros((rt, w1_ref.shape[2]), jnp.float32)
    for ky in range(3):
        xw = xs_pad[pl.ds(r0 + ky * _WPAD, rt + 8), :]
        for kx in range(3):
            xs = jax.lax.slice(xw, (kx, 0), (kx + rt, c))
            acc = acc + jnp.dot(xs, w1_ref[3 * ky + kx],
                                preferred_element_type=jnp.float32)
    hact = jnp.maximum(acc + b1_ref[0].astype(jnp.float32), 0.0)
    hact = hact.astype(jnp.bfloat16)
    out = jnp.dot(hact, w2_ref[...], preferred_element_type=jnp.float32)
    o_ref[0] = out + b2_ref[0].astype(jnp.float32)


def kernel(feats, W1, b1, W2, b2):
    B, C, H, W = feats.shape
    dim = W1.shape[0]
    co = W2.shape[0]

    RT = 768
    nR = -(-(H * _WPAD) // RT)
    Hpad = -(-(nR * RT + 2 * _WPAD + 8) // _WPAD)
    Rpad = Hpad * _WPAD
    CO2 = 128

    xc = jnp.transpose(feats.reshape(B, C, H * W), (0, 2, 1))
    w1 = jnp.transpose(W1, (2, 3, 1, 0)).reshape(9, C, dim)
    w1 = w1.astype(jnp.bfloat16)
    w2 = jnp.pad(W2[:, :, 0, 0].T, ((0, 0), (0, CO2 - co))).astype(jnp.bfloat16)
    b1r = b1.reshape(1, dim)
    b2r = jnp.pad(b2, (0, CO2 - co)).reshape(1, CO2)

    body = functools.partial(_rpn_body, rt=RT, h=H, w=W)
    out = pl.pallas_call(
        body,
        grid=(B, nR),
        in_specs=[
            pl.BlockSpec((1, H * W, C), lambda b, r: (b, 0, 0)),
            pl.BlockSpec((9, C, dim), lambda b, r: (0, 0, 0)),
            pl.BlockSpec((1, dim), lambda b, r: (0, 0)),
            pl.BlockSpec((dim, CO2), lambda b, r: (0, 0)),
            pl.BlockSpec((1, CO2), lambda b, r: (0, 0)),
        ],
        out_specs=pl.BlockSpec((1, RT, CO2), lambda b, r: (b, r, 0)),
        out_shape=jax.ShapeDtypeStruct((B, nR * RT, CO2), jnp.float32),
        scratch_shapes=[pltpu.VMEM((Rpad, C), jnp.bfloat16)],
    )(xc, w1, b1r, w2, b2r)

    out = out[:, :H * _WPAD, :].reshape(B, H, _WPAD, CO2)[:, :, :W, :co]
    return out.reshape(B, H, W, co // 6, 6)
